# speedup vs baseline: 7.2869x; 7.2869x over previous
import jax
import jax.numpy as jnp
from jax.experimental import pallas as pl

_DROP_PROB = 0.1
_BLOCK = 7
_FEAT = 224
_GAMMA = _DROP_PROB / _BLOCK**2 * (_FEAT**2 / (_FEAT - _BLOCK + 1) ** 2)
_N = _FEAT * _FEAT
_ROWS = 8 * 96
_CHUNK = 16


def _mask_kernel(u_ref, bm_ref):
    seed = (u_ref[...] < _GAMMA).astype(jnp.float32)
    zcol = jnp.zeros((_FEAT, _BLOCK - 1), jnp.float32)
    padh = jnp.concatenate([zcol, seed], axis=1)
    h = seed
    for d in range(1, _BLOCK):
        h = jnp.maximum(h, padh[:, _BLOCK - 1 - d : _BLOCK - 1 - d + _FEAT])
    zrow = jnp.zeros((_BLOCK - 1, _FEAT), jnp.float32)
    padv = jnp.concatenate([zrow, h], axis=0)
    v = h
    for d in range(1, _BLOCK):
        v = jnp.maximum(v, padv[_BLOCK - 1 - d : _BLOCK - 1 - d + _FEAT, :])
    bm_ref[...] = 1.0 - v


def _mul_kernel(bm_ref, x_ref, o_ref):
    bm = bm_ref[...]
    scale = jnp.float32(_N) / jnp.sum(bm)
    o_ref[...] = x_ref[...] * (bm * scale)


def kernel(x, u):
    bm = pl.pallas_call(
        _mask_kernel,
        out_shape=jax.ShapeDtypeStruct((_FEAT, _FEAT), jnp.float32),
    )(u)
    bm_flat = bm.reshape(1, _N)
    xf = x.reshape(_ROWS, _N)
    out = pl.pallas_call(
        _mul_kernel,
        grid=(_ROWS // _CHUNK,),
        in_specs=[
            pl.BlockSpec((1, _N), lambda i: (0, 0)),
            pl.BlockSpec((_CHUNK, _N), lambda i: (i, 0)),
        ],
        out_specs=pl.BlockSpec((_CHUNK, _N), lambda i: (i, 0)),
        out_shape=jax.ShapeDtypeStruct((_ROWS, _N), jnp.float32),
    )(bm_flat, xf)
    return out.reshape(x.shape)


# speedup vs baseline: 28.1839x; 3.8677x over previous
import jax
import jax.numpy as jnp
from jax.experimental import pallas as pl

_DROP_PROB = 0.1
_BLOCK = 7
_FEAT = 224
_GAMMA = _DROP_PROB / _BLOCK**2 * (_FEAT**2 / (_FEAT - _BLOCK + 1) ** 2)
_N = _FEAT * _FEAT
_ROWS = 8 * 96
_CHUNK = 16


def _mask_kernel(u_ref, bm_ref):
    seed = (u_ref[...] < _GAMMA).astype(jnp.float32)
    zcol = jnp.zeros((_FEAT, _BLOCK - 1), jnp.float32)
    padh = jnp.concatenate([zcol, seed], axis=1)
    h = seed
    for d in range(1, _BLOCK):
        h = jnp.maximum(h, padh[:, _BLOCK - 1 - d : _BLOCK - 1 - d + _FEAT])
    zrow = jnp.zeros((_BLOCK - 1, _FEAT), jnp.float32)
    padv = jnp.concatenate([zrow, h], axis=0)
    v = h
    for d in range(1, _BLOCK):
        v = jnp.maximum(v, padv[_BLOCK - 1 - d : _BLOCK - 1 - d + _FEAT, :])
    bm = 1.0 - v
    bm_ref[...] = bm * (jnp.float32(_N) / jnp.sum(bm))


def _mul_kernel(bm_ref, x_ref, o_ref):
    o_ref[...] = x_ref[...] * bm_ref[...]


def kernel(x, u):
    bm = pl.pallas_call(
        _mask_kernel,
        out_shape=jax.ShapeDtypeStruct((_FEAT, _FEAT), jnp.float32),
    )(u)
    b, c, h, w = x.shape
    cc = _CHUNK
    out = pl.pallas_call(
        _mul_kernel,
        grid=(b, c // cc),
        in_specs=[
            pl.BlockSpec((1, 1, h, w), lambda i, j: (0, 0, 0, 0)),
            pl.BlockSpec((1, cc, h, w), lambda i, j: (i, j, 0, 0)),
        ],
        out_specs=pl.BlockSpec((1, cc, h, w), lambda i, j: (i, j, 0, 0)),
        out_shape=jax.ShapeDtypeStruct(x.shape, jnp.float32),
    )(bm.reshape(1, 1, h, w), x)
    return out


# speedup vs baseline: 28.7557x; 1.0203x over previous
import jax
import jax.numpy as jnp
from jax.experimental import pallas as pl

_DROP_PROB = 0.1
_BLOCK = 7
_FEAT = 224
_GAMMA = _DROP_PROB / _BLOCK**2 * (_FEAT**2 / (_FEAT - _BLOCK + 1) ** 2)
_N = _FEAT * _FEAT
_ROWS = 8 * 96
_CHUNK = 32


def _mask_kernel(u_ref, bm_ref):
    seed = (u_ref[...] < _GAMMA).astype(jnp.float32)
    zcol = jnp.zeros((_FEAT, _BLOCK - 1), jnp.float32)
    padh = jnp.concatenate([zcol, seed], axis=1)
    h = seed
    for d in range(1, _BLOCK):
        h = jnp.maximum(h, padh[:, _BLOCK - 1 - d : _BLOCK - 1 - d + _FEAT])
    zrow = jnp.zeros((_BLOCK - 1, _FEAT), jnp.float32)
    padv = jnp.concatenate([zrow, h], axis=0)
    v = h
    for d in range(1, _BLOCK):
        v = jnp.maximum(v, padv[_BLOCK - 1 - d : _BLOCK - 1 - d + _FEAT, :])
    bm = 1.0 - v
    bm_ref[...] = bm * (jnp.float32(_N) / jnp.sum(bm))


def _mul_kernel(bm_ref, x_ref, o_ref):
    o_ref[...] = x_ref[...] * bm_ref[...]


def kernel(x, u):
    bm = pl.pallas_call(
        _mask_kernel,
        out_shape=jax.ShapeDtypeStruct((_FEAT, _FEAT), jnp.float32),
    )(u)
    b, c, h, w = x.shape
    cc = _CHUNK
    out = pl.pallas_call(
        _mul_kernel,
        grid=(b, c // cc),
        in_specs=[
            pl.BlockSpec((1, 1, h, w), lambda i, j: (0, 0, 0, 0)),
            pl.BlockSpec((1, cc, h, w), lambda i, j: (i, j, 0, 0)),
        ],
        out_specs=pl.BlockSpec((1, cc, h, w), lambda i, j: (i, j, 0, 0)),
        out_shape=jax.ShapeDtypeStruct(x.shape, jnp.float32),
    )(bm.reshape(1, 1, h, w), x)
    return out


# speedup vs baseline: 28.9237x; 1.0058x over previous
import jax
import jax.numpy as jnp
from jax.experimental import pallas as pl

_DROP_PROB = 0.1
_BLOCK = 7
_FEAT = 224
_GAMMA = _DROP_PROB / _BLOCK**2 * (_FEAT**2 / (_FEAT - _BLOCK + 1) ** 2)
_N = _FEAT * _FEAT
_ROWS = 8 * 96
_CHUNK = 48


def _mask_kernel(u_ref, bm_ref):
    seed = (u_ref[...] < _GAMMA).astype(jnp.float32)
    zcol = jnp.zeros((_FEAT, _BLOCK - 1), jnp.float32)
    padh = jnp.concatenate([zcol, seed], axis=1)
    h = seed
    for d in range(1, _BLOCK):
        h = jnp.maximum(h, padh[:, _BLOCK - 1 - d : _BLOCK - 1 - d + _FEAT])
    zrow = jnp.zeros((_BLOCK - 1, _FEAT), jnp.float32)
    padv = jnp.concatenate([zrow, h], axis=0)
    v = h
    for d in range(1, _BLOCK):
        v = jnp.maximum(v, padv[_BLOCK - 1 - d : _BLOCK - 1 - d + _FEAT, :])
    bm = 1.0 - v
    bm_ref[...] = bm * (jnp.float32(_N) / jnp.sum(bm))


def _mul_kernel(bm_ref, x_ref, o_ref):
    o_ref[...] = x_ref[...] * bm_ref[...]


def kernel(x, u):
    bm = pl.pallas_call(
        _mask_kernel,
        out_shape=jax.ShapeDtypeStruct((_FEAT, _FEAT), jnp.float32),
    )(u)
    b, c, h, w = x.shape
    cc = _CHUNK
    out = pl.pallas_call(
        _mul_kernel,
        grid=(b, c // cc),
        in_specs=[
            pl.BlockSpec((1, 1, h, w), lambda i, j: (0, 0, 0, 0)),
            pl.BlockSpec((1, cc, h, w), lambda i, j: (i, j, 0, 0)),
        ],
        out_specs=pl.BlockSpec((1, cc, h, w), lambda i, j: (i, j, 0, 0)),
        out_shape=jax.ShapeDtypeStruct(x.shape, jnp.float32),
    )(bm.reshape(1, 1, h, w), x)
    return out
